# baseline (speedup 1.0000x reference)
import jax
import jax.numpy as jnp
from jax.experimental import pallas as pl
from jax.experimental.pallas import tpu as pltpu


def _support_kernel(x_ref, w_ref, s_ref):
    s_ref[...] = jnp.dot(x_ref[...], w_ref[...],
                         preferred_element_type=jnp.float32)


def _spmm_kernel(adj_ref, s_ref, b_ref, out_ref):
    out_ref[...] = (
        jnp.dot(adj_ref[...], s_ref[...],
                preferred_element_type=jnp.float32)
        + b_ref[...]
    )


def _gcn_single(x, adj, W, b2):
    N, F_in = x.shape
    F_out = W.shape[1]

    support = pl.pallas_call(
        _support_kernel,
        out_shape=jax.ShapeDtypeStruct((N, F_out), jnp.float32),
    )(x, W)

    BM = min(200, N)
    return pl.pallas_call(
        _spmm_kernel,
        grid=(N // BM,),
        in_specs=[
            pl.BlockSpec((BM, N), lambda i: (i, 0)),
            pl.BlockSpec((N, F_out), lambda i: (0, 0)),
            pl.BlockSpec((1, F_out), lambda i: (0, 0)),
        ],
        out_specs=pl.BlockSpec((BM, F_out), lambda i: (i, 0)),
        out_shape=jax.ShapeDtypeStruct((N, F_out), jnp.float32),
        compiler_params=pltpu.CompilerParams(
            dimension_semantics=("parallel",)),
    )(adj, support, b2)


def kernel(input, adj, W, b):
    B, N, F_in = input.shape
    F_out = W.shape[1]
    b2 = b.reshape(1, F_out)
    outs = [_gcn_single(input[i], adj, W, b2) for i in range(B)]
    return jnp.stack(outs, axis=0)


# speedup vs baseline: 1.0415x; 1.0415x over previous
import jax
import jax.numpy as jnp
from jax.experimental import pallas as pl
from jax.experimental.pallas import tpu as pltpu


def _gcn_kernel(adj_ref, x_ref, w_ref, b_ref, out_ref, s_ref):
    @pl.when(pl.program_id(0) == 0)
    def _():
        s_ref[...] = jnp.dot(x_ref[...], w_ref[...],
                             preferred_element_type=jnp.float32)

    out_ref[...] = (
        jnp.dot(adj_ref[...], s_ref[...], preferred_element_type=jnp.float32)
        + b_ref[...]
    )


def _gcn_single(x, adj, W, b2):
    N, F_in = x.shape
    F_out = W.shape[1]

    BM = min(400, N)
    return pl.pallas_call(
        _gcn_kernel,
        grid=(N // BM,),
        in_specs=[
            pl.BlockSpec((BM, N), lambda i: (i, 0)),
            pl.BlockSpec((N, F_in), lambda i: (0, 0)),
            pl.BlockSpec((F_in, F_out), lambda i: (0, 0)),
            pl.BlockSpec((1, F_out), lambda i: (0, 0)),
        ],
        out_specs=pl.BlockSpec((BM, F_out), lambda i: (i, 0)),
        out_shape=jax.ShapeDtypeStruct((N, F_out), jnp.float32),
        scratch_shapes=[pltpu.VMEM((N, F_out), jnp.float32)],
        compiler_params=pltpu.CompilerParams(
            dimension_semantics=("arbitrary",),
            vmem_limit_bytes=64 * 1024 * 1024),
    )(adj, x, W, b2)


def kernel(input, adj, W, b):
    B, N, F_in = input.shape
    F_out = W.shape[1]
    b2 = b.reshape(1, F_out)
    outs = [_gcn_single(input[i], adj, W, b2) for i in range(B)]
    return jnp.stack(outs, axis=0)
